# baseline (speedup 1.0000x reference)
import functools

import jax
import jax.numpy as jnp
from jax import lax
from jax.experimental import pallas as pl
from jax.experimental.pallas import tpu as pltpu
from jax.experimental.pallas import tpu_sc as plsc

_NC = 2
_NS = 16
_NW = _NC * _NS
_L = 16


def _sc_scores_kernel(B, K, D, n_user_rows, n_item_rows):
    bpw = B // _NW
    ipw = bpw * K
    u_chunks = bpw // 128
    i_chunks = ipw // 128
    groups = bpw // _L

    mesh = plsc.VectorSubcoreMesh(
        core_axis_name="c", subcore_axis_name="s",
        num_cores=_NC, num_subcores=_NS)

    @functools.partial(
        pl.kernel,
        out_type=jax.ShapeDtypeStruct((B * K,), jnp.float32),
        mesh=mesh,
        scratch_types=[
            pltpu.VMEM((u_chunks, 128), jnp.int32),
            pltpu.VMEM((i_chunks, 128), jnp.int32),
            pltpu.VMEM((bpw, D), jnp.float32),
            pltpu.VMEM((ipw, D), jnp.float32),
            pltpu.VMEM((ipw,), jnp.float32),
            pltpu.SemaphoreType.DMA,
        ],
        compiler_params=pltpu.CompilerParams(
            needs_layout_passes=False, use_tc_tiling_on_sc=False),
    )
    def sc_kernel(uidx_hbm, iidx_hbm, utab_hbm, itab_hbm, out_hbm,
                  uidx_v, iidx_v, urows_v, irows_v, scores_v, sem):
        wid = lax.axis_index("s") * _NC + lax.axis_index("c")

        pltpu.sync_copy(uidx_hbm.at[pl.ds(wid * u_chunks, u_chunks)], uidx_v)
        pltpu.sync_copy(iidx_hbm.at[pl.ds(wid * i_chunks, i_chunks)], iidx_v)

        copies = []
        for c in range(u_chunks):
            copies.append(pltpu.async_copy(
                utab_hbm.at[uidx_v.at[c]],
                urows_v.at[pl.ds(c * 128, 128)], sem))
        for c in range(i_chunks):
            copies.append(pltpu.async_copy(
                itab_hbm.at[iidx_v.at[c]],
                irows_v.at[pl.ds(c * 128, 128)], sem))
        for cp in copies:
            cp.wait()

        lane = lax.iota(jnp.int32, _L)

        def group_body(g, carry):
            b16 = g * _L + lane
            j0 = b16 * K
            zeros = jnp.zeros((_L,), jnp.float32)
            accs = [[zeros, zeros] for _ in range(K)]
            for d in range(D):
                dvec = jnp.full((_L,), d, jnp.int32)
                u = plsc.load_gather(urows_v, [b16, dvec])
                for k in range(K):
                    iv = plsc.load_gather(irows_v, [j0 + k, dvec])
                    accs[k][d % 2] = accs[k][d % 2] + u * iv
            for k in range(K):
                s = accs[k][0] + accs[k][1]
                plsc.store_scatter(scores_v, [j0 + k], s)
            return carry

        lax.fori_loop(0, groups, group_body, 0)

        pltpu.sync_copy(scores_v, out_hbm.at[pl.ds(wid * ipw, ipw)])

    return sc_kernel


def _bce_mean_body(s_ref, y_ref, o_ref):
    x = s_ref[...]
    y = y_ref[...]
    loss = jnp.maximum(x, 0.0) - x * y + jnp.log1p(jnp.exp(-jnp.abs(x)))
    o_ref[0, 0] = jnp.sum(loss) * (1.0 / x.size)


def kernel(input_user, input_items, bce_label, user_table, item_table):
    B, = input_user.shape
    K = input_items.shape[1]
    D = user_table.shape[1]

    sc = _sc_scores_kernel(B, K, D, user_table.shape[0], item_table.shape[0])
    uidx = input_user.reshape(-1, 128)
    iidx = input_items.reshape(-1, 128)
    scores = sc(uidx, iidx, user_table, item_table)

    scores2d = scores.reshape(-1, 128)
    labels2d = bce_label.reshape(-1, 128)
    loss = pl.pallas_call(
        _bce_mean_body,
        out_shape=jax.ShapeDtypeStruct((1, 1), jnp.float32),
        out_specs=pl.BlockSpec(memory_space=pltpu.SMEM),
    )(scores2d, labels2d)
    return loss[0, 0]
